# Initial kernel scaffold; baseline (speedup 1.0000x reference)
import jax
import jax.numpy as jnp
from jax.experimental import pallas as pl


def kernel(edge_type, edge_feat, table):
    raise NotImplementedError("write your pallas kernel here")


# speedup vs baseline: 1.2308x; 1.2308x over previous
import functools

import jax
import jax.numpy as jnp
from jax.experimental import pallas as pl
from jax.experimental.pallas import tpu as pltpu
from jax.experimental.pallas import tpu_sc as plsc

E = 320000
D_EMB = 128
D_FEAT = 16
D_OUT = D_EMB + D_FEAT
BLK = 128


def _sc_embed_concat(table_pad, idx, feat):
    mesh = plsc.VectorSubcoreMesh(core_axis_name="core", subcore_axis_name="subcore")

    @functools.partial(
        pl.kernel,
        out_type=jax.ShapeDtypeStruct((E, D_OUT), jnp.float32),
        mesh=mesh,
        compiler_params=pltpu.CompilerParams(use_tc_tiling_on_sc=False),
    )
    def run(tab_hbm, i_hbm, f_hbm, o_hbm):
        def body(i_vmem, f_vmem, o_vmem):
            pltpu.sync_copy(tab_hbm.at[i_vmem.at[0]], o_vmem)
            @pl.loop(0, BLK)
            def _(r):
                o_vmem[r, pl.ds(D_EMB, D_FEAT)] = f_vmem[r, :]

        pltpu.emit_pipeline(
            body,
            grid=(E // BLK,),
            in_specs=[
                pl.BlockSpec((1, BLK), index_map=lambda i: (0, i)),
                pl.BlockSpec((BLK, D_FEAT), index_map=lambda i: (i, 0)),
            ],
            out_specs=[pl.BlockSpec((BLK, D_OUT), index_map=lambda i: (i, 0))],
            core_axis_name=("core", "subcore"),
            dimension_semantics=(pltpu.PARALLEL,),
        )(i_hbm, f_hbm, o_hbm)

    return run(table_pad, idx, feat)


def kernel(edge_type, edge_feat, table):
    idx = edge_type.astype(jnp.int32).reshape(1, E)
    table_pad = jnp.pad(table, ((0, 0), (0, D_FEAT)))
    return _sc_embed_concat(table_pad, idx, edge_feat)


# speedup vs baseline: 1.2346x; 1.0031x over previous
import functools

import jax
import jax.numpy as jnp
from jax.experimental import pallas as pl
from jax.experimental.pallas import tpu as pltpu
from jax.experimental.pallas import tpu_sc as plsc

E = 320000
D_EMB = 128
D_FEAT = 16
D_OUT = D_EMB + D_FEAT
BLK = 256


def _sc_embed_concat(table_pad, idx, feat):
    mesh = plsc.VectorSubcoreMesh(core_axis_name="core", subcore_axis_name="subcore")

    @functools.partial(
        pl.kernel,
        out_type=jax.ShapeDtypeStruct((E, D_OUT), jnp.float32),
        mesh=mesh,
        compiler_params=pltpu.CompilerParams(use_tc_tiling_on_sc=False),
    )
    def run(tab_hbm, i_hbm, f_hbm, o_hbm):
        def body(i_vmem, f_vmem, o_vmem):
            pltpu.sync_copy(tab_hbm.at[i_vmem.at[0]], o_vmem)
            @pl.loop(0, BLK)
            def _(r):
                o_vmem[r, pl.ds(D_EMB, D_FEAT)] = f_vmem[r, :]

        pltpu.emit_pipeline(
            body,
            grid=(E // BLK,),
            in_specs=[
                pl.BlockSpec((1, BLK), index_map=lambda i: (0, i)),
                pl.BlockSpec((BLK, D_FEAT), index_map=lambda i: (i, 0)),
            ],
            out_specs=[pl.BlockSpec((BLK, D_OUT), index_map=lambda i: (i, 0))],
            core_axis_name=("core", "subcore"),
            dimension_semantics=(pltpu.PARALLEL,),
        )(i_hbm, f_hbm, o_hbm)

    return run(table_pad, idx, feat)


def kernel(edge_type, edge_feat, table):
    idx = edge_type.astype(jnp.int32).reshape(1, E)
    table_pad = jnp.pad(table, ((0, 0), (0, D_FEAT)))
    return _sc_embed_concat(table_pad, idx, edge_feat)


# speedup vs baseline: 1.5361x; 1.2442x over previous
import functools

import jax
import jax.numpy as jnp
from jax.experimental import pallas as pl
from jax.experimental.pallas import tpu as pltpu
from jax.experimental.pallas import tpu_sc as plsc

E = 320000
D_EMB = 128
D_FEAT = 16
D_OUT = D_EMB + D_FEAT
BLK = 256


def _sc_embed_concat(table, idx, feat):
    mesh = plsc.VectorSubcoreMesh(core_axis_name="core", subcore_axis_name="subcore")

    @functools.partial(
        pl.kernel,
        out_type=jax.ShapeDtypeStruct((E, D_OUT), jnp.float32),
        mesh=mesh,
    )
    def run(tab_hbm, i_hbm, f_hbm, o_hbm):
        def emb_body(i_vmem, o_vmem):
            pltpu.sync_copy(tab_hbm.at[i_vmem.at[0]], o_vmem)

        pltpu.emit_pipeline(
            emb_body,
            grid=(E // BLK,),
            in_specs=[pl.BlockSpec((1, BLK), index_map=lambda i: (0, i))],
            out_specs=[pl.BlockSpec((BLK, D_EMB), index_map=lambda i: (i, 0))],
            core_axis_name=("core", "subcore"),
            dimension_semantics=(pltpu.PARALLEL,),
        )(i_hbm, o_hbm)

        def feat_body(f_vmem, o_vmem):
            @pl.loop(0, BLK)
            def _(r):
                o_vmem[r, :] = f_vmem[r, :]

        pltpu.emit_pipeline(
            feat_body,
            grid=(E // BLK,),
            in_specs=[pl.BlockSpec((BLK, D_FEAT), index_map=lambda i: (i, 0))],
            out_specs=[
                pl.BlockSpec((BLK, D_FEAT), index_map=lambda i: (i, D_EMB // D_FEAT))
            ],
            core_axis_name=("core", "subcore"),
            dimension_semantics=(pltpu.PARALLEL,),
        )(f_hbm, o_hbm)

    return run(table, idx, feat)


def kernel(edge_type, edge_feat, table):
    idx = edge_type.astype(jnp.int32).reshape(1, E)
    return _sc_embed_concat(table, idx, edge_feat)
